# baseline (speedup 1.0000x reference)
import jax
import jax.numpy as jnp
from jax import lax
from jax.experimental import pallas as pl
from jax.experimental.pallas import tpu as pltpu
from jax.experimental.pallas import tpu_sc as plsc

_D = 64
_L = 16


def _holo_body(idx_e_hbm, idx_r_hbm, e2_hbm, r3_hbm, out_hbm,
               idx_e_v, idx_r_v, idx_big_v, erows_v, slab_v, out_v,
               sem_e, sem_r):
    cid = lax.axis_index("c")
    sid = lax.axis_index("s")

    @pl.when(jnp.logical_and(cid == 0, sid == 0))
    def _():
        pltpu.sync_copy(idx_e_hbm, idx_e_v)
        pltpu.sync_copy(idx_r_hbm, idx_r_v)
        cp_r = pltpu.async_copy(r3_hbm.at[idx_r_v], slab_v, sem_r)
        idxv = idx_e_v[...]
        idx_big_v[...] = idxv >> 1
        cp_e = pltpu.async_copy(e2_hbm.at[idx_big_v], erows_v, sem_e)
        cp_e.wait()

        parv = (idxv & 1).astype(jnp.float32)
        ps = jnp.full((_L,), parv[0])
        po = jnp.full((_L,), parv[1])
        s_chunks = []
        o_chunks = []
        for k in range(_D // _L):
            s_lo = erows_v[0, pl.ds(k * _L, _L)]
            s_hi = erows_v[0, pl.ds(_D + k * _L, _L)]
            s_chunks.append(s_lo + ps * (s_hi - s_lo))
            o_lo = erows_v[1, pl.ds(k * _L, _L)]
            o_hi = erows_v[1, pl.ds(_D + k * _L, _L)]
            o_chunks.append(o_lo + po * (o_hi - o_lo))
        cp_r.wait()

        acc = jnp.zeros((_L,), jnp.float32)
        for c in range(_D // _L):
            t_c = jnp.zeros((_L,), jnp.float32)
            for i in range(_D):
                s_i = s_chunks[i // _L][i % _L]
                t_c = t_c + s_i * slab_v[0, i // 2,
                                         pl.ds((i % 2) * _D + c * _L, _L)]
            acc = acc + t_c * o_chunks[c]
        lanes = lax.broadcasted_iota(jnp.int32, (_L,), 0)
        dnums = lax.GatherDimensionNumbers(
            offset_dims=(), collapsed_slice_dims=(0,), start_index_map=(0,))
        for sh in (8, 4, 2, 1):
            perm = lax.gather(
                acc, (lanes ^ sh)[:, None], dnums, slice_sizes=(1,),
                mode=lax.GatherScatterMode.PROMISE_IN_BOUNDS)
            acc = acc + perm
        out_v[...] = acc
        pltpu.sync_copy(out_v, out_hbm)


def kernel(x, E_tab, R_tab):
    idx = x[0].astype(jnp.int32)
    idx_e = jnp.pad(idx[:2], (0, _L - 2))
    idx_r = idx[2:3]
    e2 = E_tab.reshape(E_tab.shape[0] // 2, 2 * _D)
    r3 = R_tab.reshape(R_tab.shape[0], _D // 2, 2 * _D)
    mesh = plsc.VectorSubcoreMesh(core_axis_name="c", subcore_axis_name="s")
    out = pl.kernel(
        _holo_body,
        out_type=jax.ShapeDtypeStruct((_L,), jnp.float32),
        mesh=mesh,
        scratch_types=[
            pltpu.VMEM((_L,), jnp.int32),
            pltpu.VMEM((1,), jnp.int32),
            pltpu.VMEM((_L,), jnp.int32),
            pltpu.VMEM((_L, 2 * _D), jnp.float32),
            pltpu.VMEM((1, _D // 2, 2 * _D), jnp.float32),
            pltpu.VMEM((_L,), jnp.float32),
            pltpu.SemaphoreType.DMA,
            pltpu.SemaphoreType.DMA,
        ],
    )(idx_e, idx_r, e2, r3)
    return out[0]


# speedup vs baseline: 1.7377x; 1.7377x over previous
import jax
import jax.numpy as jnp
from jax import lax
from jax.experimental import pallas as pl
from jax.experimental.pallas import tpu as pltpu
from jax.experimental.pallas import tpu_sc as plsc

_D = 64
_L = 16


def _holo_body(idx_hbm, e_hbm, r_hbm, out_hbm,
               idx_v, srow_v, orow_v, slab_v, out_v,
               sem_s, sem_o, sem_r):
    cid = lax.axis_index("c")
    sid = lax.axis_index("s")

    @pl.when(jnp.logical_and(cid == 0, sid == 0))
    def _():
        pltpu.sync_copy(idx_hbm, idx_v)
        idxv = idx_v[...]
        cp_r = pltpu.async_copy(r_hbm.at[idxv[2]], slab_v, sem_r)
        cp_s = pltpu.async_copy(e_hbm.at[idxv[0]], srow_v, sem_s)
        cp_o = pltpu.async_copy(e_hbm.at[idxv[1]], orow_v, sem_o)
        cp_s.wait()
        cp_o.wait()
        s_chunks = [srow_v[pl.ds(k * _L, _L)] for k in range(_D // _L)]
        o_chunks = [orow_v[pl.ds(k * _L, _L)] for k in range(_D // _L)]
        cp_r.wait()

        acc = jnp.zeros((_L,), jnp.float32)
        for c in range(_D // _L):
            t_c = jnp.zeros((_L,), jnp.float32)
            for i in range(_D):
                s_i = s_chunks[i // _L][i % _L]
                t_c = t_c + s_i * slab_v[i, pl.ds(c * _L, _L)]
            acc = acc + t_c * o_chunks[c]
        lanes = lax.broadcasted_iota(jnp.int32, (_L,), 0)
        dnums = lax.GatherDimensionNumbers(
            offset_dims=(), collapsed_slice_dims=(0,), start_index_map=(0,))
        for sh in (8, 4, 2, 1):
            perm = lax.gather(
                acc, (lanes ^ sh)[:, None], dnums, slice_sizes=(1,),
                mode=lax.GatherScatterMode.PROMISE_IN_BOUNDS)
            acc = acc + perm
        out_v[...] = acc
        pltpu.sync_copy(out_v, out_hbm)


def kernel(x, E_tab, R_tab):
    idx = jnp.pad(x[0].astype(jnp.int32), (0, _L - 3))
    mesh = plsc.VectorSubcoreMesh(core_axis_name="c", subcore_axis_name="s")
    out = pl.kernel(
        _holo_body,
        out_type=jax.ShapeDtypeStruct((_L,), jnp.float32),
        mesh=mesh,
        scratch_types=[
            pltpu.VMEM((_L,), jnp.int32),
            pltpu.VMEM((_D,), jnp.float32),
            pltpu.VMEM((_D,), jnp.float32),
            pltpu.VMEM((_D, _D), jnp.float32),
            pltpu.VMEM((_L,), jnp.float32),
            pltpu.SemaphoreType.DMA,
            pltpu.SemaphoreType.DMA,
            pltpu.SemaphoreType.DMA,
        ],
    )(idx, E_tab, R_tab)
    return out[0]
